# baseline (speedup 1.0000x reference)
import functools

import jax
import jax.numpy as jnp
from jax import lax
from jax.experimental import pallas as pl
from jax.experimental.pallas import tpu as pltpu
from jax.experimental.pallas import tpu_sc as plsc

DIM = 768
LANES = 16
NVEC = DIM // LANES
EPS = 1e-7


def _rsqrt_newton(x):
    i = lax.bitcast_convert_type(x, jnp.int32)
    y = lax.bitcast_convert_type(jnp.int32(0x5F3759DF) - (i >> 1), jnp.float32)
    for _ in range(3):
        y = y * (1.5 - 0.5 * x * y * y)
    return y


def _lane_allreduce_sum(v):
    lane = lax.iota(jnp.int32, LANES)
    for sh in (8, 4, 2, 1):
        v = v + jnp.take_along_axis(v, (lane + sh) & (LANES - 1), axis=0)
    return v


def _word_abs_sc(idx_flat, word_table, g, b, abs_table, n_tiles):
    n = idx_flat.shape[0]
    info = plsc.get_sparse_core_info()
    nw = info.num_cores * info.num_subcores
    per_w = n // nw
    chunk = 32
    n_chunks = per_w // chunk
    abs_rows = abs_table.shape[0]
    abs_n = n_tiles * abs_rows
    abs_per_w = abs_n // nw
    assert abs_per_w == per_w
    mesh = plsc.VectorSubcoreMesh(core_axis_name="c", subcore_axis_name="s")

    @functools.partial(
        pl.kernel,
        mesh=mesh,
        out_type=(
            jax.ShapeDtypeStruct((n, DIM), jnp.float32),
            jax.ShapeDtypeStruct((abs_n, DIM), jnp.float32),
        ),
        scratch_types=[
            pltpu.VMEM((per_w,), jnp.int32),
            pltpu.VMEM((chunk, DIM), jnp.float32),
            pltpu.VMEM((chunk, DIM), jnp.float32),
            pltpu.VMEM((4, 16, DIM), jnp.float32),
            pltpu.VMEM((DIM,), jnp.float32),
            pltpu.VMEM((DIM,), jnp.float32),
            pltpu.SemaphoreType.DMA,
            pltpu.SemaphoreType.DMA,
            pltpu.SemaphoreType.DMA,
            pltpu.SemaphoreType.DMA,
            pltpu.SemaphoreType.DMA,
            pltpu.SemaphoreType.DMA,
        ],
    )
    def k(idx_hbm, tab_hbm, g_hbm, b_hbm, abs_hbm, out_hbm, absout_hbm,
          idx_v, buf0, buf1, abuf, g_v, b_v, gs0, gs1, ws0, ws1, ags, aws):
        wid = lax.axis_index("s") * info.num_cores + lax.axis_index("c")
        base = wid * per_w
        pltpu.sync_copy(g_hbm, g_v)
        pltpu.sync_copy(b_hbm, b_v)
        pltpu.sync_copy(idx_hbm.at[pl.ds(base, per_w)], idx_v)
        bufs = (buf0, buf1)
        gsem = (gs0, gs1)
        wsem = (ws0, ws1)

        def process(rows_v):
            jam = 4
            @plsc.parallel_loop(0, chunk, step=jam, carry=jnp.int32(0))
            def row_body(r, cr):
                z = jnp.zeros((LANES,), jnp.float32)
                rows = [r + i for i in range(jam)]

                @plsc.parallel_loop(0, NVEC, unroll=4, carry=(z,) * (2 * jam))
                def acc_body(kk, c):
                    sl = pl.ds(kk * LANES, LANES)
                    out = []
                    for i in range(jam):
                        v = rows_v[rows[i], sl]
                        out.append(c[2 * i] + v)
                        out.append(c[2 * i + 1] + v * v)
                    return tuple(out)

                acc = acc_body
                rstd = []
                shift = []
                for i in range(jam):
                    mu = _lane_allreduce_sum(acc[2 * i]) * (1.0 / DIM)
                    var = (_lane_allreduce_sum(acc[2 * i + 1]) * (1.0 / DIM)
                           - mu * mu)
                    rs = _rsqrt_newton(var + EPS)
                    rstd.append(rs)
                    shift.append(mu * rs)

                @plsc.parallel_loop(0, NVEC, unroll=4)
                def norm_body(kk):
                    sl = pl.ds(kk * LANES, LANES)
                    gv = g_v[sl]
                    bv = b_v[sl]
                    for i in range(jam):
                        v = rows_v[rows[i], sl]
                        rows_v[rows[i], sl] = (v * rstd[i] - shift[i]) * gv + bv

                return cr

        asub = 16
        a_per_c = 2
        nabuf = 4
        abs_src0 = (wid % (abs_rows // per_w)) * per_w
        abs_gathers = [None] * (n_chunks * a_per_c)
        abs_writes = [None] * (n_chunks * a_per_c)

        def abs_issue(c):
            for j in range(a_per_c):
                a = c * a_per_c + j
                if a - nabuf >= 0:
                    abs_writes[a - nabuf].wait()
                abs_gathers[a] = pltpu.async_copy(
                    abs_hbm.at[pl.ds(abs_src0 + a * asub, asub)],
                    abuf.at[a % nabuf], ags)

        def abs_drain(c):
            for j in range(a_per_c):
                a = c * a_per_c + j
                abs_gathers[a].wait()
                abs_writes[a] = pltpu.async_copy(
                    abuf.at[a % nabuf],
                    absout_hbm.at[pl.ds(wid * per_w + a * asub, asub)], aws)

        gathers = [None] * n_chunks
        writes = [None] * n_chunks
        gathers[0] = pltpu.async_copy(
            tab_hbm.at[idx_v.at[pl.ds(0, chunk)]], bufs[0], gsem[0])
        for c in range(n_chunks):
            pb = c % 2
            nb = (c + 1) % 2
            if c + 1 < n_chunks:
                if c >= 1:
                    writes[c - 1].wait()
                gathers[c + 1] = pltpu.async_copy(
                    tab_hbm.at[idx_v.at[pl.ds((c + 1) * chunk, chunk)]],
                    bufs[nb], gsem[nb])
            abs_issue(c)
            gathers[c].wait()
            process(bufs[pb])
            abs_drain(c)
            writes[c] = pltpu.async_copy(
                bufs[pb], out_hbm.at[pl.ds(base + c * chunk, chunk)], wsem[pb])
        writes[n_chunks - 2].wait()
        writes[n_chunks - 1].wait()
        for a in range(n_chunks * a_per_c - nabuf, n_chunks * a_per_c):
            abs_writes[a].wait()

    return k(idx_flat, word_table, g, b, abs_table)


def _rel_embeddings_tc(rel_table, g, b, n_tiles):
    rows = rel_table.shape[0]
    blk = 1024
    nb = rows // blk

    def body(x_ref, g_ref, b_ref, o_ref):
        x = x_ref[...]
        mu = jnp.mean(x, axis=-1, keepdims=True)
        var = jnp.mean((x - mu) ** 2, axis=-1, keepdims=True)
        o_ref[...] = (x - mu) * lax.rsqrt(var + EPS) * g_ref[...] + b_ref[...]

    return pl.pallas_call(
        body,
        grid=(nb, n_tiles),
        in_specs=[
            pl.BlockSpec((blk, DIM), lambda j, i: (j, 0)),
            pl.BlockSpec((1, DIM), lambda j, i: (0, 0)),
            pl.BlockSpec((1, DIM), lambda j, i: (0, 0)),
        ],
        out_specs=pl.BlockSpec((blk, DIM), lambda j, i: (i * nb + j, 0)),
        out_shape=jax.ShapeDtypeStruct((n_tiles * rows, DIM), jnp.float32),
    )(rel_table, g.reshape(1, DIM), b.reshape(1, DIM))


def kernel(inputs, word_table, rel_table, abs_table, ln1_g, ln1_b, ln2_g, ln2_b):
    bsz, seq = inputs.shape
    word, abs_ = _word_abs_sc(
        inputs.reshape(-1), word_table, ln1_g, ln1_b, abs_table, bsz)
    rel = _rel_embeddings_tc(rel_table, ln2_g, ln2_b, bsz)
    return (word.reshape(bsz, seq, DIM), rel, abs_)


# speedup vs baseline: 1.1269x; 1.1269x over previous
import functools

import jax
import jax.numpy as jnp
from jax import lax
from jax.experimental import pallas as pl
from jax.experimental.pallas import tpu as pltpu
from jax.experimental.pallas import tpu_sc as plsc

DIM = 768
LANES = 16
NVEC = DIM // LANES
EPS = 1e-7


def _rsqrt_newton(x):
    i = lax.bitcast_convert_type(x, jnp.int32)
    y = lax.bitcast_convert_type(jnp.int32(0x5F3759DF) - (i >> 1), jnp.float32)
    for _ in range(3):
        y = y * (1.5 - 0.5 * x * y * y)
    return y


def _lane_allreduce_sum(v):
    lane = lax.iota(jnp.int32, LANES)
    for sh in (8, 4, 2, 1):
        v = v + jnp.take_along_axis(v, (lane + sh) & (LANES - 1), axis=0)
    return v


def _word_sc(idx_flat, word_table, g, b):
    n = idx_flat.shape[0]
    info = plsc.get_sparse_core_info()
    nw = info.num_cores * info.num_subcores
    per_w = n // nw
    chunk = 64
    n_chunks = per_w // chunk
    mesh = plsc.VectorSubcoreMesh(core_axis_name="c", subcore_axis_name="s")

    @functools.partial(
        pl.kernel,
        mesh=mesh,
        out_type=jax.ShapeDtypeStruct((n, DIM), jnp.float32),
        scratch_types=[
            pltpu.VMEM((per_w,), jnp.int32),
            pltpu.VMEM((chunk, DIM), jnp.float32),
            pltpu.VMEM((chunk, DIM), jnp.float32),
            pltpu.VMEM((DIM,), jnp.float32),
            pltpu.VMEM((DIM,), jnp.float32),
            pltpu.SemaphoreType.DMA,
            pltpu.SemaphoreType.DMA,
            pltpu.SemaphoreType.DMA,
            pltpu.SemaphoreType.DMA,
        ],
    )
    def k(idx_hbm, tab_hbm, g_hbm, b_hbm, out_hbm,
          idx_v, buf0, buf1, g_v, b_v, gs0, gs1, ws0, ws1):
        wid = lax.axis_index("s") * info.num_cores + lax.axis_index("c")
        base = wid * per_w
        pltpu.sync_copy(g_hbm, g_v)
        pltpu.sync_copy(b_hbm, b_v)
        pltpu.sync_copy(idx_hbm.at[pl.ds(base, per_w)], idx_v)
        bufs = (buf0, buf1)
        gsem = (gs0, gs1)
        wsem = (ws0, ws1)

        def process(rows_v):
            jam = 4
            @plsc.parallel_loop(0, chunk, step=jam, carry=jnp.int32(0))
            def row_body(r, cr):
                z = jnp.zeros((LANES,), jnp.float32)
                rows = [r + i for i in range(jam)]

                @plsc.parallel_loop(0, NVEC, unroll=4, carry=(z,) * (2 * jam))
                def acc_body(kk, c):
                    sl = pl.ds(kk * LANES, LANES)
                    out = []
                    for i in range(jam):
                        v = rows_v[rows[i], sl]
                        out.append(c[2 * i] + v)
                        out.append(c[2 * i + 1] + v * v)
                    return tuple(out)

                acc = acc_body
                rstd = []
                shift = []
                for i in range(jam):
                    mu = _lane_allreduce_sum(acc[2 * i]) * (1.0 / DIM)
                    var = (_lane_allreduce_sum(acc[2 * i + 1]) * (1.0 / DIM)
                           - mu * mu)
                    rs = _rsqrt_newton(var + EPS)
                    rstd.append(rs)
                    shift.append(mu * rs)

                @plsc.parallel_loop(0, NVEC, unroll=4)
                def norm_body(kk):
                    sl = pl.ds(kk * LANES, LANES)
                    gv = g_v[sl]
                    bv = b_v[sl]
                    for i in range(jam):
                        v = rows_v[rows[i], sl]
                        rows_v[rows[i], sl] = (v * rstd[i] - shift[i]) * gv + bv

                return cr

        gathers = [None] * n_chunks
        writes = [None] * n_chunks
        gathers[0] = pltpu.async_copy(
            tab_hbm.at[idx_v.at[pl.ds(0, chunk)]], bufs[0], gsem[0])
        for c in range(n_chunks):
            pb = c % 2
            nb = (c + 1) % 2
            if c + 1 < n_chunks:
                if c >= 1:
                    writes[c - 1].wait()
                gathers[c + 1] = pltpu.async_copy(
                    tab_hbm.at[idx_v.at[pl.ds((c + 1) * chunk, chunk)]],
                    bufs[nb], gsem[nb])
            gathers[c].wait()
            process(bufs[pb])
            writes[c] = pltpu.async_copy(
                bufs[pb], out_hbm.at[pl.ds(base + c * chunk, chunk)], wsem[pb])
        writes[n_chunks - 2].wait()
        writes[n_chunks - 1].wait()

    return k(idx_flat, word_table, g, b)


def _rel_embeddings_tc(rel_table, g, b, n_tiles):
    rows = rel_table.shape[0]
    blk = 1024
    nb = rows // blk

    def body(x_ref, g_ref, b_ref, o_ref):
        x = x_ref[...]
        mu = jnp.mean(x, axis=-1, keepdims=True)
        var = jnp.mean((x - mu) ** 2, axis=-1, keepdims=True)
        o_ref[...] = (x - mu) * lax.rsqrt(var + EPS) * g_ref[...] + b_ref[...]

    return pl.pallas_call(
        body,
        grid=(nb, n_tiles),
        in_specs=[
            pl.BlockSpec((blk, DIM), lambda j, i: (j, 0)),
            pl.BlockSpec((1, DIM), lambda j, i: (0, 0)),
            pl.BlockSpec((1, DIM), lambda j, i: (0, 0)),
        ],
        out_specs=pl.BlockSpec((blk, DIM), lambda j, i: (i * nb + j, 0)),
        out_shape=jax.ShapeDtypeStruct((n_tiles * rows, DIM), jnp.float32),
    )(rel_table, g.reshape(1, DIM), b.reshape(1, DIM))


def _abs_embeddings_tc(abs_table, n_tiles):
    rows = abs_table.shape[0]
    blk = 1024
    nb = rows // blk

    def body(x_ref, o_ref):
        o_ref[...] = x_ref[...]

    return pl.pallas_call(
        body,
        grid=(nb, n_tiles),
        in_specs=[pl.BlockSpec((blk, DIM), lambda j, i: (j, 0))],
        out_specs=pl.BlockSpec((blk, DIM), lambda j, i: (i * nb + j, 0)),
        out_shape=jax.ShapeDtypeStruct((n_tiles * rows, DIM), jnp.float32),
    )(abs_table)


def kernel(inputs, word_table, rel_table, abs_table, ln1_g, ln1_b, ln2_g, ln2_b):
    bsz, seq = inputs.shape
    word = _word_sc(inputs.reshape(-1), word_table, ln1_g, ln1_b)
    rel = _rel_embeddings_tc(rel_table, ln2_g, ln2_b, bsz)
    abs_ = _abs_embeddings_tc(abs_table, bsz)
    return (word.reshape(bsz, seq, DIM), rel, abs_)
